# baseline (speedup 1.0000x reference)
import functools

import jax
import jax.numpy as jnp
from jax import lax
from jax.experimental import pallas as pl
from jax.experimental.pallas import tpu as pltpu
from jax.experimental.pallas import tpu_sc as plsc

_SC_ROWS = 2048
_TC_BLK = 512

_PLANS = {
    128: ((16, 48, 56, 8), (0, 1, 1, 0)),
    96: ((16, 48, 32), (0, 1, 2)),
    64: ((16, 48), (0, 1)),
    32: ((8, 24), (0, 1)),
}


@functools.cache
def _build_sc(batch: int, hidden: int, dtype):
    info = plsc.get_sparse_core_info()
    nc, ns = info.num_cores, info.num_subcores
    nw = nc * ns
    assert batch % nw == 0
    b_per_w = batch // nw
    chunk_sizes, buf_of = _PLANS[b_per_w]
    n_chunks = len(chunk_sizes)
    nbuf = max(buf_of) + 1
    offs = []
    o = 0
    for sz in chunk_sizes:
        offs.append(o)
        o += sz
    assert o == b_per_w
    buf_rows = [
        max(chunk_sizes[c] for c in range(n_chunks) if buf_of[c] == b)
        for b in range(nbuf)
    ]
    mesh = plsc.VectorSubcoreMesh(core_axis_name="c", subcore_axis_name="s")

    spmem_chunk = 1 if b_per_w == 128 else None

    scratch = [pltpu.VMEM((b_per_w,), jnp.int32)]
    scratch += [pltpu.VMEM((buf_rows[b], hidden), dtype) for b in range(nbuf)]
    scratch += [pltpu.SemaphoreType.DMA] * (2 * nbuf)
    if spmem_chunk is not None:
        scratch += [
            pltpu.VMEM_SHARED((ns, chunk_sizes[spmem_chunk], hidden), dtype),
            pltpu.SemaphoreType.DMA,
            pltpu.SemaphoreType.DMA,
        ]

    @functools.partial(
        pl.kernel,
        mesh=mesh,
        out_type=jax.ShapeDtypeStruct((batch, hidden), dtype),
        scratch_types=scratch,
    )
    def emb(table_hbm, idx_hbm, out_hbm, idx_v, *rest):
        bufs = rest[:nbuf]
        gsems = rest[nbuf : 2 * nbuf]
        ssems = rest[2 * nbuf : 3 * nbuf]
        sid = lax.axis_index("s")
        wid = sid * nc + lax.axis_index("c")
        base = wid * b_per_w
        pltpu.sync_copy(idx_hbm.at[pl.ds(base, b_per_w)], idx_v)

        def gather(c):
            b = buf_of[c]
            dst = bufs[b]
            if chunk_sizes[c] != buf_rows[b]:
                dst = dst.at[pl.ds(0, chunk_sizes[c])]
            return pltpu.async_copy(
                table_hbm.at[idx_v.at[pl.ds(offs[c], chunk_sizes[c])]],
                dst, gsems[b],
            )

        def store(c):
            b = buf_of[c]
            src = bufs[b]
            if chunk_sizes[c] != buf_rows[b]:
                src = src.at[pl.ds(0, chunk_sizes[c])]
            return pltpu.async_copy(
                src, out_hbm.at[pl.ds(base + offs[c], chunk_sizes[c])],
                ssems[b],
            )

        if spmem_chunk is not None:
            spm, psem, qsem = rest[3 * nbuf :]
            my_spm = spm.at[sid]
            g0, g1 = gather(0), gather(1)
            g0.wait()
            s0 = store(0)
            g1.wait()
            src1 = bufs[1].at[pl.ds(0, chunk_sizes[1])]
            ph = pltpu.async_copy(src1, my_spm, psem)
            ph.wait()
            g2 = gather(2)
            q1 = pltpu.async_copy(
                my_spm,
                out_hbm.at[pl.ds(base + offs[1], chunk_sizes[1])],
                qsem,
            )
            s0.wait()
            g3 = gather(3)
            g2.wait()
            s2 = store(2)
            g3.wait()
            s3 = store(3)
            q1.wait()
            s2.wait()
            s3.wait()
            return

        gh, sh = {}, {}
        waited = set()
        for c in range(n_chunks):
            if buf_of[c] == c:
                gh[c] = gather(c)
        for c in range(n_chunks):
            if buf_of[c] != c:
                owner = buf_of[c]
                sh[owner].wait()
                waited.add(owner)
                gh[c] = gather(c)
            gh[c].wait()
            sh[c] = store(c)
        for c in range(n_chunks):
            if c not in waited:
                sh[c].wait()

    return emb


@functools.cache
def _build_tc(t_rows: int, hidden: int, v_pad: int):
    blk = _TC_BLK
    assert t_rows % blk == 0
    grid = t_rows // blk

    def body(lbl_ref, tab_ref, out_ref):
        lbl = lbl_ref[0]
        iota_v = lax.broadcasted_iota(jnp.int32, (v_pad, blk), 0)
        oh_t = (iota_v == lbl).astype(jnp.bfloat16)
        out_ref[...] = lax.dot_general(
            oh_t, tab_ref[...],
            dimension_numbers=(((0,), (0,)), ((), ())),
            preferred_element_type=jnp.float32,
        )

    return pl.pallas_call(
        body,
        grid=(grid,),
        in_specs=[
            pl.BlockSpec((1, 1, blk), lambda i: (i, 0, 0)),
            pl.BlockSpec((v_pad, hidden), lambda i: (0, 0)),
        ],
        out_specs=pl.BlockSpec((blk, hidden), lambda i: (i, 0)),
        out_shape=jax.ShapeDtypeStruct((t_rows, hidden), jnp.float32),
    )


def kernel(labels, embedding_table):
    _, hidden = embedding_table.shape
    batch = labels.shape[0]
    return _build_sc(batch, hidden, embedding_table.dtype)(
        embedding_table, labels
    )


# speedup vs baseline: 1.0781x; 1.0781x over previous
import functools

import jax
import jax.numpy as jnp
from jax import lax
from jax.experimental import pallas as pl
from jax.experimental.pallas import tpu as pltpu
from jax.experimental.pallas import tpu_sc as plsc

_PLANS = {
    128: ((16, 48, 56, 8), (0, 1, 2, 0)),
    96: ((16, 48, 32), (0, 1, 2)),
    64: ((16, 48), (0, 1)),
    32: ((8, 24), (0, 1)),
}


@functools.cache
def _build_sc(batch: int, hidden: int, dtype):
    info = plsc.get_sparse_core_info()
    nc, ns = info.num_cores, info.num_subcores
    nw = nc * ns
    assert batch % nw == 0
    b_per_w = batch // nw
    chunk_sizes, buf_of = _PLANS[b_per_w]
    n_chunks = len(chunk_sizes)
    nbuf = max(buf_of) + 1
    offs = []
    o = 0
    for sz in chunk_sizes:
        offs.append(o)
        o += sz
    assert o == b_per_w
    buf_rows = [
        max(chunk_sizes[c] for c in range(n_chunks) if buf_of[c] == b)
        for b in range(nbuf)
    ]
    mesh = plsc.VectorSubcoreMesh(core_axis_name="c", subcore_axis_name="s")

    scratch = [pltpu.VMEM((b_per_w,), jnp.int32)]
    scratch += [pltpu.VMEM((buf_rows[b], hidden), dtype) for b in range(nbuf)]
    scratch += [pltpu.SemaphoreType.DMA] * (2 * nbuf)

    @functools.partial(
        pl.kernel,
        mesh=mesh,
        out_type=jax.ShapeDtypeStruct((batch, hidden), dtype),
        scratch_types=scratch,
    )
    def emb(table_hbm, idx_hbm, out_hbm, idx_v, *rest):
        bufs = rest[:nbuf]
        gsems = rest[nbuf : 2 * nbuf]
        ssems = rest[2 * nbuf :]
        wid = lax.axis_index("s") * nc + lax.axis_index("c")
        base = wid * b_per_w
        pltpu.sync_copy(idx_hbm.at[pl.ds(base, b_per_w)], idx_v)

        def gather(c):
            b = buf_of[c]
            dst = bufs[b]
            if chunk_sizes[c] != buf_rows[b]:
                dst = dst.at[pl.ds(0, chunk_sizes[c])]
            return pltpu.async_copy(
                table_hbm.at[idx_v.at[pl.ds(offs[c], chunk_sizes[c])]],
                dst, gsems[b],
            )

        def store(c):
            b = buf_of[c]
            src = bufs[b]
            if chunk_sizes[c] != buf_rows[b]:
                src = src.at[pl.ds(0, chunk_sizes[c])]
            return pltpu.async_copy(
                src, out_hbm.at[pl.ds(base + offs[c], chunk_sizes[c])],
                ssems[b],
            )

        gh, sh = {}, {}
        waited = set()
        for c in range(n_chunks):
            if buf_of[c] == c:
                gh[c] = gather(c)
        for c in range(n_chunks):
            if buf_of[c] != c:
                owner = buf_of[c]
                sh[owner].wait()
                waited.add(owner)
                gh[c] = gather(c)
            gh[c].wait()
            sh[c] = store(c)
        for c in range(n_chunks):
            if c not in waited:
                sh[c].wait()

    return emb


def kernel(labels, embedding_table):
    _, hidden = embedding_table.shape
    return _build_sc(labels.shape[0], hidden, embedding_table.dtype)(
        embedding_table, labels
    )


# speedup vs baseline: 1.0790x; 1.0009x over previous
import functools

import jax
import jax.numpy as jnp
from jax import lax
from jax.experimental import pallas as pl
from jax.experimental.pallas import tpu as pltpu
from jax.experimental.pallas import tpu_sc as plsc

_PLANS = {
    128: ((16, 48, 56, 8), (0, 1, 2, 0)),
    96: ((16, 48, 32), (0, 1, 2)),
    64: ((16, 48), (0, 1)),
    32: ((8, 24), (0, 1)),
}


@functools.cache
def _build_sc(batch: int, hidden: int, dtype):
    info = plsc.get_sparse_core_info()
    nc, ns = info.num_cores, info.num_subcores
    nw = nc * ns
    assert batch % nw == 0
    b_per_w = batch // nw
    chunk_sizes, buf_of = _PLANS[b_per_w]
    n_chunks = len(chunk_sizes)
    nbuf = max(buf_of) + 1
    offs = []
    o = 0
    for sz in chunk_sizes:
        offs.append(o)
        o += sz
    assert o == b_per_w
    buf_rows = [
        max(chunk_sizes[c] for c in range(n_chunks) if buf_of[c] == b)
        for b in range(nbuf)
    ]
    mesh = plsc.VectorSubcoreMesh(core_axis_name="c", subcore_axis_name="s")

    scratch = [pltpu.VMEM((b_per_w,), jnp.int32)]
    scratch += [pltpu.VMEM((buf_rows[b], hidden), dtype) for b in range(nbuf)]
    scratch += [pltpu.SemaphoreType.DMA] * (2 * nbuf)

    @functools.partial(
        pl.kernel,
        mesh=mesh,
        out_type=jax.ShapeDtypeStruct((batch, hidden), dtype),
        scratch_types=scratch,
    )
    def emb(table_hbm, idx_hbm, out_hbm, idx_v, *rest):
        bufs = rest[:nbuf]
        gsems = rest[nbuf : 2 * nbuf]
        ssems = rest[2 * nbuf :]
        wid = lax.axis_index("c") * ns + lax.axis_index("s")
        base = wid * b_per_w
        pltpu.sync_copy(idx_hbm.at[pl.ds(base, b_per_w)], idx_v)

        def gather(c):
            b = buf_of[c]
            dst = bufs[b]
            if chunk_sizes[c] != buf_rows[b]:
                dst = dst.at[pl.ds(0, chunk_sizes[c])]
            return pltpu.async_copy(
                table_hbm.at[idx_v.at[pl.ds(offs[c], chunk_sizes[c])]],
                dst, gsems[b],
            )

        def store(c):
            b = buf_of[c]
            src = bufs[b]
            if chunk_sizes[c] != buf_rows[b]:
                src = src.at[pl.ds(0, chunk_sizes[c])]
            return pltpu.async_copy(
                src, out_hbm.at[pl.ds(base + offs[c], chunk_sizes[c])],
                ssems[b],
            )

        gh, sh = {}, {}
        waited = set()
        for c in range(n_chunks):
            if buf_of[c] == c:
                gh[c] = gather(c)
        for c in range(n_chunks):
            if buf_of[c] != c:
                owner = buf_of[c]
                sh[owner].wait()
                waited.add(owner)
                gh[c] = gather(c)
            gh[c].wait()
            sh[c] = store(c)
        for c in range(n_chunks):
            if c not in waited:
                sh[c].wait()

    return emb


def kernel(labels, embedding_table):
    _, hidden = embedding_table.shape
    return _build_sc(labels.shape[0], hidden, embedding_table.dtype)(
        embedding_table, labels
    )
